# baseline (speedup 1.0000x reference)
import functools

import jax
import jax.numpy as jnp
from jax import lax
from jax.experimental import pallas as pl
from jax.experimental.pallas import tpu as pltpu
from jax.experimental.pallas import tpu_sc as plsc

_B = 1024
_NNEG = 50
_DIM = 64
_NPAIR = 6
_NHEAD = 5

_NP = 56
_NC = 2
_NS = 16
_NW = _NC * _NS
_BPW = _B // _NW


def _gates_tc(x, gw, gb):
    def body(x_ref, w_ref, b_ref, o_ref):
        o_ref[...] = jnp.tanh(
            jnp.dot(x_ref[...], w_ref[...], preferred_element_type=jnp.float32,
                    precision=lax.Precision.HIGHEST)
            + b_ref[...]
        )
    return pl.pallas_call(
        body,
        out_shape=jax.ShapeDtypeStruct((_B, _NPAIR * _DIM), jnp.float32),
    )(x, gw, gb)


_V = 100000
_NBLK = _V // 128
_TAIL = _V - _NBLK * 128
_KST = 25


def _make_conv_kernel():
    mesh = plsc.VectorSubcoreMesh(core_axis_name="c", subcore_axis_name="s")

    @functools.partial(
        pl.kernel,
        out_type=[jax.ShapeDtypeStruct((_V * _DIM,), jnp.float32)
                  for _ in range(_NHEAD)],
        mesh=mesh,
        scratch_types=[
            pltpu.VMEM((_NHEAD, _DIM, 128), jnp.float32),
            pltpu.VMEM((128 * _DIM,), jnp.float32),
            pltpu.VMEM((128 * _DIM,), jnp.float32),
            pltpu.VMEM((128 * _DIM,), jnp.float32),
            pltpu.VMEM((128 * _DIM,), jnp.float32),
            pltpu.VMEM((128 * _DIM,), jnp.float32),
            pltpu.SemaphoreType.DMA,
            pltpu.SemaphoreType.DMA,
            pltpu.SemaphoreType.DMA,
            pltpu.SemaphoreType.DMA,
            pltpu.SemaphoreType.DMA,
            pltpu.SemaphoreType.DMA,
            pltpu.SemaphoreType.DMA,
            pltpu.SemaphoreType.DMA,
            pltpu.SemaphoreType.DMA,
            pltpu.SemaphoreType.DMA,
        ],
        compiler_params=pltpu.CompilerParams(
            needs_layout_passes=False, use_tc_tiling_on_sc=True),
    )
    def conv(e0, e1, e2, e3, e4, tail_hbm, o0, o1, o2, o3, o4, inb,
             ob0, ob1, ob2, ob3, ob4,
             si0, si1, si2, si3, si4, so0, so1, so2, so3, so4):
        wid = lax.axis_index("s") * _NC + lax.axis_index("c")
        embs = (e0, e1, e2, e3, e4)
        outs = (o0, o1, o2, o3, o4)
        si = (si0, si1, si2, si3, si4)
        outb = (ob0, ob1, ob2, ob3, ob4)
        so = (so0, so1, so2, so3, so4)
        lanes = lax.iota(jnp.int32, 16)
        rowv = [c * 16 + lanes for c in range(4)]

        def gblk(k):
            return jnp.minimum(wid + _NW * k, _NBLK - 1)

        def wait_i(h):
            pltpu.make_async_copy(embs[h].at[:, pl.ds(0, 128)], inb.at[h],
                                  si[h]).wait()

        def wait_o(h):
            pltpu.make_async_copy(outb[h], outs[h].at[pl.ds(0, 128 * _DIM)],
                                  so[h]).wait()

        for h in range(_NHEAD):
            pltpu.async_copy(outb[h],
                             outs[h].at[pl.ds(wid * 128 * _DIM, 128 * _DIM)],
                             so[h])
            pltpu.async_copy(embs[h].at[:, pl.ds(gblk(0) * 128, 128)],
                             inb.at[h], si[h])

        def body(k, carry):
            for h in range(_NHEAD):
                wait_i(h)

                def rbody(r, _, h=h):
                    for c in range(4):
                        vals = plsc.load_gather(
                            inb.at[h], [rowv[c], jnp.full((16,), 0,
                                                          jnp.int32) + r])
                        outb[h][pl.ds(r * _DIM + c * 16, 16)] = vals
                    return 0

                lax.fori_loop(0, 128, rbody, 0, unroll=4)
                gn = gblk(k + 1)
                pltpu.async_copy(embs[h].at[:, pl.ds(gn * 128, 128)],
                                 inb.at[h], si[h])
                wait_o(h)
                g = gblk(k)
                pltpu.async_copy(outb[h],
                                 outs[h].at[pl.ds(g * 128 * _DIM, 128 * _DIM)],
                                 so[h])
            return carry

        lax.fori_loop(0, _KST, body, 0)

        for h in range(_NHEAD):
            wait_i(h)
            wait_o(h)

        @pl.when(wid == 0)
        def _tail():
            for h in range(_NHEAD):
                pltpu.sync_copy(tail_hbm.at[pl.ds(h * _TAIL * _DIM,
                                                  _TAIL * _DIM)],
                                outb[0].at[pl.ds(0, _TAIL * _DIM)])
                pltpu.sync_copy(outb[0].at[pl.ds(0, _TAIL * _DIM)],
                                outs[h].at[pl.ds(_NBLK * 128 * _DIM,
                                                 _TAIL * _DIM)])

    return conv


_conv_kernel = _make_conv_kernel()


def _make_sc_kernel():
    mesh = plsc.VectorSubcoreMesh(core_axis_name="c", subcore_axis_name="s")

    @functools.partial(
        pl.kernel,
        out_type=jax.ShapeDtypeStruct((_B, _NP), jnp.float32),
        mesh=mesh,
        scratch_types=[
            pltpu.VMEM((6, _NNEG, _BPW), jnp.int32),
            pltpu.VMEM((2, _NHEAD, 64), jnp.int32),
            pltpu.VMEM((_BPW, _NPAIR * _DIM), jnp.float32),
            pltpu.VMEM((2, _NHEAD, _NP, _DIM), jnp.float32),
            pltpu.VMEM((64, 16), jnp.float32),
            pltpu.VMEM((2, 64), jnp.float32),
            pltpu.SemaphoreType.DMA,
            pltpu.SemaphoreType.DMA,
            pltpu.SemaphoreType.DMA,
            pltpu.SemaphoreType.DMA,
        ],
        compiler_params=pltpu.CompilerParams(
            needs_layout_passes=False, use_tc_tiling_on_sc=False),
    )
    def sc(labels_hbm, gates_hbm, t0, t1, t2, t3, t4, out_hbm,
           lab_v, idx_v, gate_v, rows_v, accs_v, score_v, sem_g0, sem_g1,
           sem_o0, sem_o1):
        wid = lax.axis_index("s") * _NC + lax.axis_index("c")
        base = wid * _BPW
        tables = (t0, t1, t2, t3, t4)
        sem_g = (sem_g0, sem_g1)
        sem_o = (sem_o0, sem_o1)
        zero16 = jnp.zeros((16,), jnp.float32)
        lanes = lax.iota(jnp.int32, 16)
        for r in range(_NP, 64):
            accs_v[r] = zero16

        pltpu.sync_copy(labels_hbm.at[:, :, pl.ds(base, _BPW)], lab_v)
        pltpu.sync_copy(gates_hbm.at[pl.ds(base, _BPW)], gate_v)

        nvecs = [jnp.minimum(q * 16 + lanes, _NNEG - 1) for q in range(4)]

        def prefetch(i, s):
            ivec = jnp.full((16,), 0, jnp.int32) + i
            for h in range(_NHEAD):
                hvec = jnp.full((16,), h, jnp.int32)
                for q in range(4):
                    vals = plsc.load_gather(lab_v, [hvec, nvecs[q], ivec])
                    idx_v[s, h, pl.ds(q * 16, 16)] = vals
            for h in range(_NHEAD):
                pltpu.async_copy(tables[h].at[idx_v.at[s, h, pl.ds(0, _NP)]],
                                 rows_v.at[s, h], sem_g[s])

        def drain_g(s):
            pltpu.make_async_copy(labels_hbm.at[0], rows_v.at[s],
                                  sem_g[s]).wait()

        def wait_o(s):
            pltpu.make_async_copy(score_v.at[s, pl.ds(0, _NP)],
                                  out_hbm.at[base + s], sem_o[s]).wait()

        def consume(i, s):
            drain_g(s)
            for c in range(4):
                g0 = gate_v[i, pl.ds(0 * 64 + c * 16, 16)]
                g1 = gate_v[i, pl.ds(1 * 64 + c * 16, 16)]
                g2 = gate_v[i, pl.ds(2 * 64 + c * 16, 16)]
                g3 = gate_v[i, pl.ds(3 * 64 + c * 16, 16)]
                g4 = gate_v[i, pl.ds(4 * 64 + c * 16, 16)]
                g5 = gate_v[i, pl.ds(5 * 64 + c * 16, 16)]
                sl = pl.ds(c * 16, 16)

                def neg_body(n, _, c=c, sl=sl, s=s, g0=g0, g1=g1, g2=g2,
                             g3=g3, g4=g4, g5=g5):
                    pred = rows_v[s, 0, n, sl]
                    perm = rows_v[s, 1, n, sl]
                    prim = rows_v[s, 2, n, sl]
                    sec = rows_v[s, 3, n, sl]
                    reo = rows_v[s, 4, n, sl]
                    a1 = pred * g0 + reo * g3
                    a2 = pred * g1 + reo * g4
                    a3 = pred * g2 + reo * g5
                    contrib = a1 * perm + a2 * prim + a3 * sec
                    if c == 0:
                        accs_v[n] = contrib
                    else:
                        plsc.addupdate(accs_v.at[n], contrib)
                    return 0

                lax.fori_loop(0, _NP, neg_body, 0, unroll=8)

            wait_o(s)
            for grp in range(4):
                tot = zero16
                for j in range(16):
                    v = jnp.sum(accs_v[grp * 16 + j])
                    tot = jnp.where(lanes == j, v, tot)
                score_v[s, pl.ds(grp * 16, 16)] = tot
            pltpu.async_copy(score_v.at[s, pl.ds(0, _NP)],
                             out_hbm.at[base + i], sem_o[s])

        pltpu.async_copy(score_v.at[0, pl.ds(0, _NP)], out_hbm.at[base + 0],
                         sem_o0)
        pltpu.async_copy(score_v.at[1, pl.ds(0, _NP)], out_hbm.at[base + 1],
                         sem_o1)
        prefetch(0, 0)

        def body(k, carry):
            i0 = 2 * k
            prefetch(i0 + 1, 1)
            consume(i0, 0)
            prefetch(jnp.minimum(i0 + 2, _BPW - 1), 0)
            consume(i0 + 1, 1)
            return carry

        lax.fori_loop(0, _BPW // 2, body, 0)

        drain_g(0)
        wait_o(0)
        wait_o(1)

    return sc


_sc_kernel = _make_sc_kernel()


def kernel(x, neg_labels, emb_predictor, emb_cf_perm, emb_cf_primary,
           emb_cf_secondary, emb_reorder,
           gw_predictor__cf_perm, gb_predictor__cf_perm,
           gw_predictor__cf_primary, gb_predictor__cf_primary,
           gw_predictor__cf_secondary, gb_predictor__cf_secondary,
           gw_reorder__cf_perm, gb_reorder__cf_perm,
           gw_reorder__cf_primary, gb_reorder__cf_primary,
           gw_reorder__cf_secondary, gb_reorder__cf_secondary):
    gw = jnp.concatenate(
        [gw_predictor__cf_perm, gw_predictor__cf_primary,
         gw_predictor__cf_secondary, gw_reorder__cf_perm,
         gw_reorder__cf_primary, gw_reorder__cf_secondary], axis=1)
    gb = jnp.concatenate(
        [gb_predictor__cf_perm, gb_predictor__cf_primary,
         gb_predictor__cf_secondary, gb_reorder__cf_perm,
         gb_reorder__cf_primary, gb_reorder__cf_secondary], axis=0)
    gates = _gates_tc(x, gw, gb.reshape(1, _NPAIR * _DIM))

    lab = jnp.transpose(neg_labels, (2, 1, 0))

    _embs = [emb_predictor, emb_cf_perm, emb_cf_primary, emb_cf_secondary,
             emb_reorder]
    tail = jnp.concatenate([e[_NBLK * 128:].reshape(-1) for e in _embs])
    tlin = _conv_kernel(*[jnp.transpose(e) for e in _embs], tail)
    tabs = [t.reshape(_V, _DIM) for t in tlin]

    score = _sc_kernel(lab, gates, *tabs)
    return score[:, :_NNEG]


# speedup vs baseline: 2.3550x; 2.3550x over previous
import functools

import jax
import jax.numpy as jnp
from jax import lax
from jax.experimental import pallas as pl
from jax.experimental.pallas import tpu as pltpu
from jax.experimental.pallas import tpu_sc as plsc

_B = 1024
_NNEG = 50
_DIM = 64
_NPAIR = 6
_NHEAD = 5

_NP = 56
_NC = 2
_NS = 16
_NW = _NC * _NS
_BPW = _B // _NW


def _gates_tc(x, gw, gb):
    def body(x_ref, w_ref, b_ref, o_ref):
        o_ref[...] = jnp.tanh(
            jnp.dot(x_ref[...], w_ref[...], preferred_element_type=jnp.float32,
                    precision=lax.Precision.HIGHEST)
            + b_ref[...]
        )
    return pl.pallas_call(
        body,
        out_shape=jax.ShapeDtypeStruct((_B, _NPAIR * _DIM), jnp.float32),
    )(x, gw, gb)


def _make_sc_kernel():
    mesh = plsc.VectorSubcoreMesh(core_axis_name="c", subcore_axis_name="s")

    @functools.partial(
        pl.kernel,
        out_type=jax.ShapeDtypeStruct((_B, _NP), jnp.float32),
        mesh=mesh,
        scratch_types=[
            pltpu.VMEM((6, _NNEG, _BPW), jnp.int32),
            pltpu.VMEM((2, _NHEAD, 64), jnp.int32),
            pltpu.VMEM((_BPW, _NPAIR * _DIM), jnp.float32),
            pltpu.VMEM((2, _NHEAD, _NP, _DIM), jnp.float32),
            pltpu.VMEM((64, 16), jnp.float32),
            pltpu.VMEM((2, 64), jnp.float32),
            pltpu.SemaphoreType.DMA,
            pltpu.SemaphoreType.DMA,
            pltpu.SemaphoreType.DMA,
            pltpu.SemaphoreType.DMA,
        ],
        compiler_params=pltpu.CompilerParams(
            needs_layout_passes=False, use_tc_tiling_on_sc=False),
    )
    def sc(labels_hbm, gates_hbm, t0, t1, t2, t3, t4, out_hbm,
           lab_v, idx_v, gate_v, rows_v, accs_v, score_v, sem_g0, sem_g1,
           sem_o0, sem_o1):
        wid = lax.axis_index("s") * _NC + lax.axis_index("c")
        base = wid * _BPW
        tables = (t0, t1, t2, t3, t4)
        sem_g = (sem_g0, sem_g1)
        sem_o = (sem_o0, sem_o1)
        zero16 = jnp.zeros((16,), jnp.float32)
        lanes = lax.iota(jnp.int32, 16)
        for r in range(_NP, 64):
            accs_v[r] = zero16

        pltpu.sync_copy(labels_hbm.at[:, :, pl.ds(base, _BPW)], lab_v)
        pltpu.sync_copy(gates_hbm.at[pl.ds(base, _BPW)], gate_v)

        nvecs = [jnp.minimum(q * 16 + lanes, _NNEG - 1) for q in range(4)]

        def prefetch(i, s):
            ivec = jnp.full((16,), 0, jnp.int32) + i
            for h in range(_NHEAD):
                hvec = jnp.full((16,), h, jnp.int32)
                for q in range(4):
                    vals = plsc.load_gather(lab_v, [hvec, nvecs[q], ivec])
                    idx_v[s, h, pl.ds(q * 16, 16)] = vals
            for h in range(_NHEAD):
                pltpu.async_copy(tables[h].at[idx_v.at[s, h, pl.ds(0, _NP)]],
                                 rows_v.at[s, h], sem_g[s])

        def drain_g(s):
            pltpu.make_async_copy(labels_hbm.at[0], rows_v.at[s],
                                  sem_g[s]).wait()

        def wait_o(s):
            pltpu.make_async_copy(score_v.at[s, pl.ds(0, _NP)],
                                  out_hbm.at[base + s], sem_o[s]).wait()

        def consume(i, s):
            drain_g(s)
            for c in range(4):
                g0 = gate_v[i, pl.ds(0 * 64 + c * 16, 16)]
                g1 = gate_v[i, pl.ds(1 * 64 + c * 16, 16)]
                g2 = gate_v[i, pl.ds(2 * 64 + c * 16, 16)]
                g3 = gate_v[i, pl.ds(3 * 64 + c * 16, 16)]
                g4 = gate_v[i, pl.ds(4 * 64 + c * 16, 16)]
                g5 = gate_v[i, pl.ds(5 * 64 + c * 16, 16)]
                sl = pl.ds(c * 16, 16)

                def neg_body(n, _, c=c, sl=sl, s=s, g0=g0, g1=g1, g2=g2,
                             g3=g3, g4=g4, g5=g5):
                    pred = rows_v[s, 0, n, sl]
                    perm = rows_v[s, 1, n, sl]
                    prim = rows_v[s, 2, n, sl]
                    sec = rows_v[s, 3, n, sl]
                    reo = rows_v[s, 4, n, sl]
                    a1 = pred * g0 + reo * g3
                    a2 = pred * g1 + reo * g4
                    a3 = pred * g2 + reo * g5
                    contrib = a1 * perm + a2 * prim + a3 * sec
                    if c == 0:
                        accs_v[n] = contrib
                    else:
                        plsc.addupdate(accs_v.at[n], contrib)
                    return 0

                lax.fori_loop(0, _NP, neg_body, 0, unroll=8)

            wait_o(s)
            for grp in range(4):
                tot = zero16
                for j in range(16):
                    v = jnp.sum(accs_v[grp * 16 + j])
                    tot = jnp.where(lanes == j, v, tot)
                score_v[s, pl.ds(grp * 16, 16)] = tot
            pltpu.async_copy(score_v.at[s, pl.ds(0, _NP)],
                             out_hbm.at[base + i], sem_o[s])

        pltpu.async_copy(score_v.at[0, pl.ds(0, _NP)], out_hbm.at[base + 0],
                         sem_o0)
        pltpu.async_copy(score_v.at[1, pl.ds(0, _NP)], out_hbm.at[base + 1],
                         sem_o1)
        prefetch(0, 0)

        def body(k, carry):
            i0 = 2 * k
            prefetch(i0 + 1, 1)
            consume(i0, 0)
            prefetch(jnp.minimum(i0 + 2, _BPW - 1), 0)
            consume(i0 + 1, 1)
            return carry

        lax.fori_loop(0, _BPW // 2, body, 0)

        drain_g(0)
        wait_o(0)
        wait_o(1)

    return sc


_sc_kernel = _make_sc_kernel()


def kernel(x, neg_labels, emb_predictor, emb_cf_perm, emb_cf_primary,
           emb_cf_secondary, emb_reorder,
           gw_predictor__cf_perm, gb_predictor__cf_perm,
           gw_predictor__cf_primary, gb_predictor__cf_primary,
           gw_predictor__cf_secondary, gb_predictor__cf_secondary,
           gw_reorder__cf_perm, gb_reorder__cf_perm,
           gw_reorder__cf_primary, gb_reorder__cf_primary,
           gw_reorder__cf_secondary, gb_reorder__cf_secondary):
    gw = jnp.concatenate(
        [gw_predictor__cf_perm, gw_predictor__cf_primary,
         gw_predictor__cf_secondary, gw_reorder__cf_perm,
         gw_reorder__cf_primary, gw_reorder__cf_secondary], axis=1)
    gb = jnp.concatenate(
        [gb_predictor__cf_perm, gb_predictor__cf_primary,
         gb_predictor__cf_secondary, gb_reorder__cf_perm,
         gb_reorder__cf_primary, gb_reorder__cf_secondary], axis=0)
    gates = _gates_tc(x, gw, gb.reshape(1, _NPAIR * _DIM))

    lab = jnp.transpose(neg_labels, (2, 1, 0))

    score = _sc_kernel(lab, gates, emb_predictor, emb_cf_perm,
                       emb_cf_primary, emb_cf_secondary, emb_reorder)
    return score[:, :_NNEG]
